# baseline (speedup 1.0000x reference)
import jax
import jax.numpy as jnp
from jax.experimental import pallas as pl
from jax.experimental.pallas import tpu as pltpu

_N = 10000
_BM1 = 288
_BM2 = 2048
_F8MAX = 448.0


def _layer1_body(adj_ref, xk_ref, xm_ref, wl_ref, bl_ref, wr_ref, br_ref,
                 h_ref, q_ref, hq_ref):
    a = adj_ref[...]
    q_ref[...] = a.astype(jnp.float8_e4m3fn)
    s = jax.lax.dot_general(
        a.astype(jnp.bfloat16), xk_ref[...].astype(jnp.bfloat16),
        (((1,), (0,)), ((), ())), preferred_element_type=jnp.float32)
    out = jax.lax.dot_general(
        s, wl_ref[...], (((1,), (1,)), ((), ())),
        preferred_element_type=jnp.float32) + bl_ref[...]
    out = out + jax.lax.dot_general(
        xm_ref[...], wr_ref[...], (((1,), (1,)), ((), ())),
        preferred_element_type=jnp.float32) + br_ref[...]
    denom = jnp.maximum(jnp.sum(jnp.abs(out), axis=1, keepdims=True), 1e-12)
    h = jnp.maximum(out / denom, 0.0)
    h_ref[...] = h
    hq_ref[...] = (h * _F8MAX).astype(jnp.float8_e4m3fn)


def _layer2_body(q_ref, hq_ref, hm_ref, wl_ref, bl_ref, wr_ref, br_ref,
                 out_ref):
    s = jax.lax.dot_general(
        q_ref[...], hq_ref[...],
        (((1,), (0,)), ((), ())), preferred_element_type=jnp.float32)
    z1 = s * jnp.float32(1.0 / _F8MAX)
    out = jax.lax.dot_general(
        z1, wl_ref[...], (((1,), (1,)), ((), ())),
        preferred_element_type=jnp.float32) + bl_ref[...]
    out = out + jax.lax.dot_general(
        hm_ref[...], wr_ref[...], (((1,), (1,)), ((), ())),
        preferred_element_type=jnp.float32) + br_ref[...]
    m = jnp.max(out, axis=1, keepdims=True)
    e = out - m
    lse = jnp.log(jnp.sum(jnp.exp(e), axis=1, keepdims=True))
    out_ref[...] = e - lse


def kernel(x, block, Wl0, bl0, Wr0, br0, Wl1, bl1, Wr1, br1):
    n, f = x.shape

    h, q, hq = pl.pallas_call(
        _layer1_body,
        grid=(pl.cdiv(_N, _BM1),),
        in_specs=[
            pl.BlockSpec((_BM1, _N), lambda i: (i, 0)),
            pl.BlockSpec((n, f), lambda i: (0, 0)),
            pl.BlockSpec((_BM1, f), lambda i: (i, 0)),
            pl.BlockSpec((f, f), lambda i: (0, 0)),
            pl.BlockSpec((1, f), lambda i: (0, 0)),
            pl.BlockSpec((f, f), lambda i: (0, 0)),
            pl.BlockSpec((1, f), lambda i: (0, 0)),
        ],
        out_specs=[
            pl.BlockSpec((_BM1, f), lambda i: (i, 0)),
            pl.BlockSpec((_BM1, _N), lambda i: (i, 0)),
            pl.BlockSpec((_BM1, f), lambda i: (i, 0)),
        ],
        out_shape=[
            jax.ShapeDtypeStruct((n, f), jnp.float32),
            jax.ShapeDtypeStruct((_N, _N), jnp.float8_e4m3fn),
            jax.ShapeDtypeStruct((n, f), jnp.float8_e4m3fn),
        ],
        compiler_params=pltpu.CompilerParams(
            dimension_semantics=("parallel",)),
    )(block, x, x, Wl0, bl0.reshape(1, f), Wr0, br0.reshape(1, f))

    return pl.pallas_call(
        _layer2_body,
        grid=(pl.cdiv(_N, _BM2),),
        in_specs=[
            pl.BlockSpec((_BM2, _N), lambda i: (i, 0)),
            pl.BlockSpec((n, f), lambda i: (0, 0)),
            pl.BlockSpec((_BM2, f), lambda i: (i, 0)),
            pl.BlockSpec((f, f), lambda i: (0, 0)),
            pl.BlockSpec((1, f), lambda i: (0, 0)),
            pl.BlockSpec((f, f), lambda i: (0, 0)),
            pl.BlockSpec((1, f), lambda i: (0, 0)),
        ],
        out_specs=pl.BlockSpec((_BM2, f), lambda i: (i, 0)),
        out_shape=jax.ShapeDtypeStruct((n, f), jnp.float32),
        compiler_params=pltpu.CompilerParams(
            dimension_semantics=("parallel",),
            vmem_limit_bytes=100 * 1024 * 1024),
    )(q, hq, h, Wl1, bl1.reshape(1, f), Wr1, br1.reshape(1, f))


# speedup vs baseline: 1.0323x; 1.0323x over previous
import jax
import jax.numpy as jnp
from jax.experimental import pallas as pl
from jax.experimental.pallas import tpu as pltpu

_N = 10000
_BM1 = 512
_BM2 = 1024
_F8MAX = 448.0


def _layer1_body(adj_ref, xk_ref, xm_ref, wl_ref, bl_ref, wr_ref, br_ref,
                 h_ref, q_ref, hq_ref):
    a = adj_ref[...]
    q_ref[...] = a.astype(jnp.float8_e4m3fn)
    s = jax.lax.dot_general(
        a.astype(jnp.bfloat16), xk_ref[...].astype(jnp.bfloat16),
        (((1,), (0,)), ((), ())), preferred_element_type=jnp.float32)
    out = jax.lax.dot_general(
        s, wl_ref[...], (((1,), (1,)), ((), ())),
        preferred_element_type=jnp.float32) + bl_ref[...]
    out = out + jax.lax.dot_general(
        xm_ref[...], wr_ref[...], (((1,), (1,)), ((), ())),
        preferred_element_type=jnp.float32) + br_ref[...]
    denom = jnp.maximum(jnp.sum(jnp.abs(out), axis=1, keepdims=True), 1e-12)
    h = jnp.maximum(out / denom, 0.0)
    h_ref[...] = h
    hq_ref[...] = (h * _F8MAX).astype(jnp.float8_e4m3fn)


def _layer2_body(q_ref, hq_ref, hm_ref, wl_ref, bl_ref, wr_ref, br_ref,
                 out_ref):
    s = jax.lax.dot_general(
        q_ref[...], hq_ref[...],
        (((1,), (0,)), ((), ())), preferred_element_type=jnp.float32)
    z1 = s * jnp.float32(1.0 / _F8MAX)
    out = jax.lax.dot_general(
        z1, wl_ref[...], (((1,), (1,)), ((), ())),
        preferred_element_type=jnp.float32) + bl_ref[...]
    out = out + jax.lax.dot_general(
        hm_ref[...], wr_ref[...], (((1,), (1,)), ((), ())),
        preferred_element_type=jnp.float32) + br_ref[...]
    m = jnp.max(out, axis=1, keepdims=True)
    e = out - m
    lse = jnp.log(jnp.sum(jnp.exp(e), axis=1, keepdims=True))
    out_ref[...] = e - lse


def kernel(x, block, Wl0, bl0, Wr0, br0, Wl1, bl1, Wr1, br1):
    n, f = x.shape

    h, q, hq = pl.pallas_call(
        _layer1_body,
        grid=(pl.cdiv(_N, _BM1),),
        in_specs=[
            pl.BlockSpec((_BM1, _N), lambda i: (i, 0)),
            pl.BlockSpec((n, f), lambda i: (0, 0)),
            pl.BlockSpec((_BM1, f), lambda i: (i, 0)),
            pl.BlockSpec((f, f), lambda i: (0, 0)),
            pl.BlockSpec((1, f), lambda i: (0, 0)),
            pl.BlockSpec((f, f), lambda i: (0, 0)),
            pl.BlockSpec((1, f), lambda i: (0, 0)),
        ],
        out_specs=[
            pl.BlockSpec((_BM1, f), lambda i: (i, 0)),
            pl.BlockSpec((_BM1, _N), lambda i: (i, 0)),
            pl.BlockSpec((_BM1, f), lambda i: (i, 0)),
        ],
        out_shape=[
            jax.ShapeDtypeStruct((n, f), jnp.float32),
            jax.ShapeDtypeStruct((_N, _N), jnp.float8_e4m3fn),
            jax.ShapeDtypeStruct((n, f), jnp.float8_e4m3fn),
        ],
        compiler_params=pltpu.CompilerParams(
            dimension_semantics=("parallel",),
            vmem_limit_bytes=100 * 1024 * 1024),
    )(block, x, x, Wl0, bl0.reshape(1, f), Wr0, br0.reshape(1, f))

    return pl.pallas_call(
        _layer2_body,
        grid=(pl.cdiv(_N, _BM2),),
        in_specs=[
            pl.BlockSpec((_BM2, _N), lambda i: (i, 0)),
            pl.BlockSpec((n, f), lambda i: (0, 0)),
            pl.BlockSpec((_BM2, f), lambda i: (i, 0)),
            pl.BlockSpec((f, f), lambda i: (0, 0)),
            pl.BlockSpec((1, f), lambda i: (0, 0)),
            pl.BlockSpec((f, f), lambda i: (0, 0)),
            pl.BlockSpec((1, f), lambda i: (0, 0)),
        ],
        out_specs=pl.BlockSpec((_BM2, f), lambda i: (i, 0)),
        out_shape=jax.ShapeDtypeStruct((n, f), jnp.float32),
        compiler_params=pltpu.CompilerParams(
            dimension_semantics=("parallel",),
            vmem_limit_bytes=100 * 1024 * 1024),
    )(q, hq, h, Wl1, bl1.reshape(1, f), Wr1, br1.reshape(1, f))
